# baseline (speedup 1.0000x reference)
import jax
import jax.numpy as jnp
from jax.experimental import pallas as pl
from jax.experimental.pallas import tpu as pltpu

_C = 3200
_K = 8


def _make_body(n_full, n_tail):
    def body(e0, e1, r0, r1, out_e, out_r, a, b, o, ta, tb, to, ra, rb, ro,
             sa, sb, so, sta, stb, sto, sra, srb, sro):
        cra = pltpu.make_async_copy(r0, ra, sra)
        crb = pltpu.make_async_copy(r1, rb, srb)
        cra.start()
        crb.start()
        if n_tail:
            tail0 = n_full * _C
            cta = pltpu.make_async_copy(e0.at[:, pl.ds(tail0, n_tail)], ta, sta)
            ctb = pltpu.make_async_copy(e1.at[:, pl.ds(tail0, n_tail)], tb, stb)
            cta.start()
            ctb.start()

        in_a = {}
        in_b = {}
        out_o = {}

        def start_in(i):
            s = i % _K
            sl = pl.ds(i * _C, _C)
            in_a[i] = pltpu.make_async_copy(e0.at[:, sl], a.at[s], sa.at[s])
            in_b[i] = pltpu.make_async_copy(e1.at[:, sl], b.at[s], sb.at[s])
            in_a[i].start()
            in_b[i].start()

        for i in range(min(_K, n_full)):
            start_in(i)

        for i in range(n_full):
            s = i % _K
            in_a[i].wait()
            in_b[i].wait()
            if i >= _K:
                out_o[i - _K].wait()
            o[s] = a[s] + b[s]
            out_o[i] = pltpu.make_async_copy(
                o.at[s], out_e.at[:, pl.ds(i * _C, _C)], so.at[s])
            out_o[i].start()
            j = i + _K
            if j < n_full:
                start_in(j)

        if n_tail:
            cta.wait()
            ctb.wait()
            to[...] = ta[...] + tb[...]
            cto = pltpu.make_async_copy(
                to, out_e.at[:, pl.ds(n_full * _C, n_tail)], sto)
            cto.start()

        cra.wait()
        crb.wait()
        ro[...] = ra[...] + rb[...]
        cro = pltpu.make_async_copy(ro, out_r, sro)
        cro.start()
        for i in range(max(n_full - _K, 0), n_full):
            out_o[i].wait()
        if n_tail:
            cto.wait()
        cro.wait()

    return body


def kernel(inputs, ent_embeds_0, rel_embeds_0, ent_embeds_1, rel_embeds_1):
    n_ent, d_ent = ent_embeds_0.shape
    n_rel, d_rel = rel_embeds_0.shape
    e0t, e1t = ent_embeds_0.T, ent_embeds_1.T
    r0t, r1t = rel_embeds_0.T, rel_embeds_1.T
    n_full, n_tail = divmod(n_ent, _C)
    body = _make_body(n_full, n_tail)
    hbm = pl.BlockSpec(memory_space=pl.ANY)
    tail_cols = n_tail if n_tail else 1
    out_et, out_rt = pl.pallas_call(
        body,
        in_specs=[hbm, hbm, hbm, hbm],
        out_specs=[hbm, hbm],
        out_shape=[
            jax.ShapeDtypeStruct((d_ent, n_ent), ent_embeds_0.dtype),
            jax.ShapeDtypeStruct((d_rel, n_rel), rel_embeds_0.dtype),
        ],
        scratch_shapes=[
            pltpu.VMEM((_K, d_ent, _C), ent_embeds_0.dtype),
            pltpu.VMEM((_K, d_ent, _C), ent_embeds_0.dtype),
            pltpu.VMEM((_K, d_ent, _C), ent_embeds_0.dtype),
            pltpu.VMEM((d_ent, tail_cols), ent_embeds_0.dtype),
            pltpu.VMEM((d_ent, tail_cols), ent_embeds_0.dtype),
            pltpu.VMEM((d_ent, tail_cols), ent_embeds_0.dtype),
            pltpu.VMEM((d_rel, n_rel), rel_embeds_0.dtype),
            pltpu.VMEM((d_rel, n_rel), rel_embeds_0.dtype),
            pltpu.VMEM((d_rel, n_rel), rel_embeds_0.dtype),
            pltpu.SemaphoreType.DMA((_K,)),
            pltpu.SemaphoreType.DMA((_K,)),
            pltpu.SemaphoreType.DMA((_K,)),
            pltpu.SemaphoreType.DMA,
            pltpu.SemaphoreType.DMA,
            pltpu.SemaphoreType.DMA,
            pltpu.SemaphoreType.DMA,
            pltpu.SemaphoreType.DMA,
            pltpu.SemaphoreType.DMA,
        ],
    )(e0t, e1t, r0t, r1t)
    return (out_et.T, out_rt.T)


# speedup vs baseline: 1.0079x; 1.0079x over previous
import jax
import jax.numpy as jnp
from jax.experimental import pallas as pl
from jax.experimental.pallas import tpu as pltpu

_BC = 16384


def _add_kernel(e0, e1, r0, r1, out_e, out_r):
    out_e[...] = e0[...] + e1[...]

    @pl.when(pl.program_id(0) == 0)
    def _():
        out_r[...] = r0[...] + r1[...]


def kernel(inputs, ent_embeds_0, rel_embeds_0, ent_embeds_1, rel_embeds_1):
    n_ent, d_ent = ent_embeds_0.shape
    n_rel, d_rel = rel_embeds_0.shape
    e0t, e1t = ent_embeds_0.T, ent_embeds_1.T
    r0t, r1t = rel_embeds_0.T, rel_embeds_1.T
    grid = (pl.cdiv(n_ent, _BC),)
    ent_spec = pl.BlockSpec((d_ent, _BC), lambda i: (0, i))
    rel_spec = pl.BlockSpec((d_rel, n_rel), lambda i: (0, 0))
    out_et, out_rt = pl.pallas_call(
        _add_kernel,
        grid=grid,
        in_specs=[ent_spec, ent_spec, rel_spec, rel_spec],
        out_specs=[ent_spec, rel_spec],
        out_shape=[
            jax.ShapeDtypeStruct((d_ent, n_ent), ent_embeds_0.dtype),
            jax.ShapeDtypeStruct((d_rel, n_rel), rel_embeds_0.dtype),
        ],
        compiler_params=pltpu.CompilerParams(
            dimension_semantics=("parallel",),
        ),
    )(e0t, e1t, r0t, r1t)
    return (out_et.T, out_rt.T)
